# Initial kernel scaffold; baseline (speedup 1.0000x reference)
import jax
import jax.numpy as jnp
from jax.experimental import pallas as pl


def kernel(x, edge_index, W1, b1, W2, b2, W3, b3):
    raise NotImplementedError("write your pallas kernel here")


# speedup vs baseline: 6.6889x; 6.6889x over previous
import jax
import jax.numpy as jnp
from jax import lax
from jax.experimental import pallas as pl
from jax.experimental.pallas import tpu as pltpu
from jax.experimental.pallas import tpu_sc as plsc

N = 10000
E = 320000
F = 128

NC = 2
NS = 16

NP = 10240
NPAD = NP - N
FH = F // NC
NPT = NP // NS
RCH = 64
NRC = NPT // RCH
B = 128
SUBG = 8
EPT = 20480
NGRP = EPT // (B * SUBG)
EPAD = EPT - E // NS

_mesh = plsc.VectorSubcoreMesh(
    core_axis_name="c", subcore_axis_name="s", num_cores=NC, num_subcores=NS
)
_sc_params = pltpu.CompilerParams(needs_layout_passes=False, use_tc_tiling_on_sc=False)


def _rsqrt16(d):
    i = lax.bitcast_convert_type(d, jnp.int32)
    i = jnp.int32(0x5F3759DF) - lax.shift_right_logical(i, 1)
    y = lax.bitcast_convert_type(i, jnp.float32)
    for _ in range(4):
        y = y * (1.5 - 0.5 * d * y * y)
    return jnp.where(d > 0.0, y, 0.0)


def _deg_dis_body(row_hbm, dis_hbm, rbuf, ones, dbuf, obuf, dsh, sem):
    c = lax.axis_index("c")
    s = lax.axis_index("s")

    @pl.when(c == 0)
    def _prep():
        def f_ones(i, _):
            ones[pl.ds(i * 16, 16)] = jnp.full((16,), 1.0, jnp.float32)
            return 0

        lax.fori_loop(0, B // 16, f_ones, 0)

        def f_zero(i, _):
            dbuf[pl.ds(i * 16, 16)] = jnp.zeros((16,), jnp.float32)
            return 0

        lax.fori_loop(0, NPT // 16, f_zero, 0)
        pltpu.sync_copy(dbuf, dsh.at[pl.ds(s * NPT, NPT)])

    plsc.subcore_barrier()

    @pl.when(c == 0)
    def _scatter():
        def grp(g, _):
            pltpu.sync_copy(row_hbm.at[s, pl.ds(g * SUBG, SUBG), :], rbuf)
            for jj in range(SUBG):
                pltpu.sync_copy(ones, dsh.at[rbuf.at[jj]], add=True)
            return 0

        lax.fori_loop(0, NGRP, grp, 0)

    plsc.subcore_barrier()

    @pl.when(c == 0)
    def _rsqrt():
        pltpu.sync_copy(dsh.at[pl.ds(s * NPT, NPT)], dbuf)

        def grp(i, _):
            obuf[pl.ds(i * 16, 16)] = _rsqrt16(dbuf[pl.ds(i * 16, 16)])
            return 0

        lax.fori_loop(0, NPT // 16, grp, 0)
        pltpu.sync_copy(obuf, dis_hbm.at[s])


_deg_dis = pl.kernel(
    _deg_dis_body,
    out_type=jax.ShapeDtypeStruct((NS, NPT), jnp.float32),
    mesh=_mesh,
    scratch_types=[
        pltpu.VMEM((SUBG, B), jnp.int32),
        pltpu.VMEM((B,), jnp.float32),
        pltpu.VMEM((NPT,), jnp.float32),
        pltpu.VMEM((NPT,), jnp.float32),
        pltpu.VMEM_SHARED((NP,), jnp.float32),
        pltpu.SemaphoreType.DMA,
    ],
    compiler_params=_sc_params,
)


def _make_prop(recur):
    scale = -2.0 if recur else -1.0

    def body(*refs):
        if recur:
            (t_hbm, prev_hbm, row_hbm, col_hbm, dis_hbm, out_hbm,
             tsh, ash, rbuf, cbuf, tbuf, sbuf, abuf, obuf, gbuf, disb,
             sem) = refs
        else:
            (t_hbm, row_hbm, col_hbm, dis_hbm, out_hbm,
             tsh, ash, rbuf, cbuf, tbuf, sbuf, abuf, obuf, gbuf, disb,
             sem) = refs
            prev_hbm = None

        c = lax.axis_index("c")
        s = lax.axis_index("s")
        base = pl.multiple_of(s * NPT, RCH)

        pltpu.sync_copy(dis_hbm.at[s], disb)

        for k in range(NRC):
            r0 = pl.multiple_of(base + k * RCH, RCH)
            rl = k * RCH
            p0 = pl.multiple_of(s * (NPT // 2) + k * (RCH // 2), RCH // 2)
            pltpu.sync_copy(t_hbm.at[c, pl.ds(p0, RCH // 2), :], tbuf)

            def srow(p, _, rl=rl):
                for half in range(2):
                    sp = plsc.load_gather(
                        disb, [jnp.broadcast_to(rl + 2 * p + half, (16,))]
                    )
                    for f in range(FH // 16):
                        o = half * FH + f * 16
                        sbuf[2 * p + half, pl.ds(f * 16, 16)] = (
                            tbuf[p, pl.ds(o, 16)] * sp
                        )
                return 0

            lax.fori_loop(0, RCH // 2, srow, 0)
            pltpu.sync_copy(sbuf, tsh.at[pl.ds(r0, RCH), :])

        def zrow(r, _):
            for f in range(FH // 16):
                sbuf[r, pl.ds(f * 16, 16)] = jnp.zeros((16,), jnp.float32)
            return 0

        lax.fori_loop(0, RCH, zrow, 0)
        for k in range(NRC):
            pltpu.sync_copy(sbuf, ash.at[pl.ds(base + k * RCH, RCH), :])

        plsc.subcore_barrier()

        def grp(g, _):
            pltpu.sync_copy(row_hbm.at[s, pl.ds(g * SUBG, SUBG), :], rbuf)
            pltpu.sync_copy(col_hbm.at[s, pl.ds(g * SUBG, SUBG), :], cbuf)
            for jj in range(SUBG):
                pltpu.async_copy(tsh.at[rbuf.at[jj]], gbuf, sem).wait()
                pltpu.sync_copy(gbuf, ash.at[cbuf.at[jj]], add=True)
            return 0

        lax.fori_loop(0, NGRP, grp, 0)

        plsc.subcore_barrier()

        for k in range(NRC):
            r0 = pl.multiple_of(base + k * RCH, RCH)
            rl = k * RCH
            p0 = pl.multiple_of(s * (NPT // 2) + k * (RCH // 2), RCH // 2)
            pltpu.sync_copy(ash.at[pl.ds(r0, RCH), :], abuf)
            if recur:
                pltpu.sync_copy(prev_hbm.at[c, pl.ds(p0, RCH // 2), :], tbuf)

            def erow(p, _, rl=rl):
                for half in range(2):
                    sp = plsc.load_gather(
                        disb, [jnp.broadcast_to(rl + 2 * p + half, (16,))]
                    )
                    sp = sp * scale
                    for f in range(FH // 16):
                        o = half * FH + f * 16
                        v = abuf[2 * p + half, pl.ds(f * 16, 16)] * sp
                        if recur:
                            v = v - tbuf[p, pl.ds(o, 16)]
                        obuf[p, pl.ds(o, 16)] = v
                return 0

            lax.fori_loop(0, RCH // 2, erow, 0)
            pltpu.sync_copy(obuf, out_hbm.at[c, pl.ds(p0, RCH // 2), :])

    return pl.kernel(
        body,
        out_type=jax.ShapeDtypeStruct((NC, NP // 2, F), jnp.float32),
        mesh=_mesh,
        scratch_types=[
            pltpu.VMEM_SHARED((NP, FH), jnp.float32),
            pltpu.VMEM_SHARED((NP, FH), jnp.float32),
            pltpu.VMEM((SUBG, B), jnp.int32),
            pltpu.VMEM((SUBG, B), jnp.int32),
            pltpu.VMEM((RCH // 2, F), jnp.float32),
            pltpu.VMEM((RCH, FH), jnp.float32),
            pltpu.VMEM((RCH, FH), jnp.float32),
            pltpu.VMEM((RCH // 2, F), jnp.float32),
            pltpu.VMEM((B, FH), jnp.float32),
            pltpu.VMEM((NPT,), jnp.float32),
            pltpu.SemaphoreType.DMA,
        ],
        compiler_params=_sc_params,
    )


_prop_plain = _make_prop(False)
_prop_recur = _make_prop(True)


def _tc_layer(t0, t1, t2, t3, W, b, last):
    H = W.shape[2]
    BR = 512
    G = NP // BR

    def body(t0r, t1r, t2r, t3r, wr, br, outr):
        acc = br[...] * jnp.float32(1.0)
        for k, tr in enumerate((t0r, t1r, t2r, t3r)):
            acc = acc + jnp.dot(
                tr[...], wr[k], preferred_element_type=jnp.float32
            )
        if last:
            m = jnp.max(acc, axis=1, keepdims=True)
            z = acc - m
            lse = jnp.log(jnp.sum(jnp.exp(z), axis=1, keepdims=True))
            outr[...] = z - lse
        else:
            outr[...] = jnp.maximum(acc, 0.0)

    spec = pl.BlockSpec((BR, F), lambda i: (i, 0))
    return pl.pallas_call(
        body,
        out_shape=jax.ShapeDtypeStruct((NP, H), jnp.float32),
        grid=(G,),
        in_specs=[
            spec, spec, spec, spec,
            pl.BlockSpec((4, F, H), lambda i: (0, 0, 0)),
            pl.BlockSpec((1, H), lambda i: (0, 0)),
        ],
        out_specs=pl.BlockSpec((BR, H), lambda i: (i, 0)),
    )(_unsplit(t0), _unsplit(t1), _unsplit(t2), _unsplit(t3), W, b)


def _split(x2d):
    return jnp.stack(
        [x2d[:, :FH].reshape(NP // 2, F), x2d[:, FH:].reshape(NP // 2, F)]
    )


def _unsplit(slabs):
    return jnp.concatenate(
        [slabs[0].reshape(NP, FH), slabs[1].reshape(NP, FH)], axis=1
    )


def _pad_edges(idx):
    per = idx.reshape(NS, E // NS)
    fill = N + (jnp.arange(EPAD, dtype=jnp.int32) % NPAD)
    fill = jnp.broadcast_to(fill, (NS, EPAD))
    return jnp.concatenate([per, fill], axis=1).reshape(NS, EPT // B, B)


def kernel(x, edge_index, W1, b1, W2, b2, W3, b3):
    row2 = _pad_edges(edge_index[0])
    col2 = _pad_edges(edge_index[1])

    dis = _deg_dis(row2)
    xp = _split(jnp.pad(x, ((0, NP - N), (0, 0))))

    def cheb(t, W, b, last):
        t0 = t
        t1 = _prop_plain(t0, row2, col2, dis)
        t2 = _prop_recur(t1, t0, row2, col2, dis)
        t3 = _prop_recur(t2, t1, row2, col2, dis)
        return _tc_layer(t0, t1, t2, t3, W, b.reshape(1, -1), last)

    h = cheb(xp, W1, b1, False)
    h = cheb(_split(h), W2, b2, False)
    out = cheb(_split(h), W3, b3, True)
    return out[:N]


# speedup vs baseline: 8.1588x; 1.2198x over previous
import jax
import jax.numpy as jnp
from jax import lax
from jax.experimental import pallas as pl
from jax.experimental.pallas import tpu as pltpu
from jax.experimental.pallas import tpu_sc as plsc

N = 10000
E = 320000
F = 128

NC = 2
NS = 16

NP = 10240
NPAD = NP - N
FH = F // NC
NPT = NP // NS
RCH = 64
NRC = NPT // RCH
B = 128
SUBG = 8
EPT = 20480
NGRP = EPT // (B * SUBG)
EPAD = EPT - E // NS

_mesh = plsc.VectorSubcoreMesh(
    core_axis_name="c", subcore_axis_name="s", num_cores=NC, num_subcores=NS
)
_sc_params = pltpu.CompilerParams(needs_layout_passes=False, use_tc_tiling_on_sc=False)


def _rsqrt16(d):
    i = lax.bitcast_convert_type(d, jnp.int32)
    i = jnp.int32(0x5F3759DF) - lax.shift_right_logical(i, 1)
    y = lax.bitcast_convert_type(i, jnp.float32)
    for _ in range(4):
        y = y * (1.5 - 0.5 * d * y * y)
    return jnp.where(d > 0.0, y, 0.0)


def _deg_dis_body(row_hbm, dis_hbm, rbuf, ones, dbuf, obuf, dsh, sem):
    c = lax.axis_index("c")
    s = lax.axis_index("s")

    @pl.when(c == 0)
    def _prep():
        def f_ones(i, _):
            ones[pl.ds(i * 16, 16)] = jnp.full((16,), 1.0, jnp.float32)
            return 0

        lax.fori_loop(0, B // 16, f_ones, 0)

        def f_zero(i, _):
            dbuf[pl.ds(i * 16, 16)] = jnp.zeros((16,), jnp.float32)
            return 0

        lax.fori_loop(0, NPT // 16, f_zero, 0)
        pltpu.sync_copy(dbuf, dsh.at[pl.ds(s * NPT, NPT)])

    plsc.subcore_barrier()

    @pl.when(c == 0)
    def _scatter():
        def grp(g, _):
            pltpu.sync_copy(row_hbm.at[s, pl.ds(g * SUBG, SUBG), :], rbuf)
            for jj in range(SUBG):
                pltpu.sync_copy(ones, dsh.at[rbuf.at[jj]], add=True)
            return 0

        lax.fori_loop(0, NGRP, grp, 0)

    plsc.subcore_barrier()

    @pl.when(c == 0)
    def _rsqrt():
        pltpu.sync_copy(dsh.at[pl.ds(s * NPT, NPT)], dbuf)

        def grp(i, _):
            obuf[pl.ds(i * 16, 16)] = _rsqrt16(dbuf[pl.ds(i * 16, 16)])
            return 0

        lax.fori_loop(0, NPT // 16, grp, 0)
        pltpu.sync_copy(obuf, dis_hbm.at[s])


_deg_dis = pl.kernel(
    _deg_dis_body,
    out_type=jax.ShapeDtypeStruct((NS, NPT), jnp.float32),
    mesh=_mesh,
    scratch_types=[
        pltpu.VMEM((SUBG, B), jnp.int32),
        pltpu.VMEM((B,), jnp.float32),
        pltpu.VMEM((NPT,), jnp.float32),
        pltpu.VMEM((NPT,), jnp.float32),
        pltpu.VMEM_SHARED((NP,), jnp.float32),
        pltpu.SemaphoreType.DMA,
    ],
    compiler_params=_sc_params,
)


def _make_prop(recur):
    scale = -2.0 if recur else -1.0

    def body(*refs):
        if recur:
            (t_hbm, prev_hbm, row_hbm, col_hbm, dis_hbm, out_hbm,
             tsh, ash, rbuf, cbuf, tbuf, sbuf, abuf, obuf, gbuf, gbuf2,
             disb, sem, sem2, ssem, ssem2) = refs
        else:
            (t_hbm, row_hbm, col_hbm, dis_hbm, out_hbm,
             tsh, ash, rbuf, cbuf, tbuf, sbuf, abuf, obuf, gbuf, gbuf2,
             disb, sem, sem2, ssem, ssem2) = refs
            prev_hbm = None

        c = lax.axis_index("c")
        s = lax.axis_index("s")
        base = pl.multiple_of(s * NPT, RCH)

        pltpu.sync_copy(dis_hbm.at[s], disb)

        for k in range(NRC):
            r0 = pl.multiple_of(base + k * RCH, RCH)
            rl = k * RCH
            p0 = pl.multiple_of(s * (NPT // 2) + k * (RCH // 2), RCH // 2)
            pltpu.sync_copy(t_hbm.at[c, pl.ds(p0, RCH // 2), :], tbuf)

            def srow(p, _, rl=rl):
                for half in range(2):
                    sp = plsc.load_gather(
                        disb, [jnp.broadcast_to(rl + 2 * p + half, (16,))]
                    )
                    for f in range(FH // 16):
                        o = half * FH + f * 16
                        sbuf[2 * p + half, pl.ds(f * 16, 16)] = (
                            tbuf[p, pl.ds(o, 16)] * sp
                        )
                return 0

            lax.fori_loop(0, RCH // 2, srow, 0)
            pltpu.sync_copy(sbuf, tsh.at[pl.ds(r0, RCH), :])

        def zrow(r, _):
            for f in range(FH // 16):
                sbuf[r, pl.ds(f * 16, 16)] = jnp.zeros((16,), jnp.float32)
            return 0

        lax.fori_loop(0, RCH, zrow, 0)
        for k in range(NRC):
            pltpu.sync_copy(sbuf, ash.at[pl.ds(base + k * RCH, RCH), :])

        plsc.subcore_barrier()

        def grp(g, _):
            pltpu.sync_copy(row_hbm.at[s, pl.ds(g * SUBG, SUBG), :], rbuf)
            pltpu.sync_copy(col_hbm.at[s, pl.ds(g * SUBG, SUBG), :], cbuf)
            bufs = (gbuf, gbuf2)
            gsems = (sem, sem2)
            ssems = (ssem, ssem2)
            g_desc = [None] * SUBG
            s_desc = [None] * SUBG
            g_desc[0] = pltpu.async_copy(
                tsh.at[rbuf.at[0]], bufs[0], gsems[0]
            )
            for jj in range(SUBG):
                b = jj % 2
                g_desc[jj].wait()
                s_desc[jj] = pltpu.async_copy(
                    bufs[b], ash.at[cbuf.at[jj]], ssems[b], add=True
                )
                if jj + 1 < SUBG:
                    if jj >= 1:
                        s_desc[jj - 1].wait()
                    g_desc[jj + 1] = pltpu.async_copy(
                        tsh.at[rbuf.at[jj + 1]], bufs[(jj + 1) % 2],
                        gsems[(jj + 1) % 2],
                    )
            s_desc[SUBG - 2].wait()
            s_desc[SUBG - 1].wait()
            return 0

        lax.fori_loop(0, NGRP, grp, 0)

        plsc.subcore_barrier()

        for k in range(NRC):
            r0 = pl.multiple_of(base + k * RCH, RCH)
            rl = k * RCH
            p0 = pl.multiple_of(s * (NPT // 2) + k * (RCH // 2), RCH // 2)
            pltpu.sync_copy(ash.at[pl.ds(r0, RCH), :], abuf)
            if recur:
                pltpu.sync_copy(prev_hbm.at[c, pl.ds(p0, RCH // 2), :], tbuf)

            def erow(p, _, rl=rl):
                for half in range(2):
                    sp = plsc.load_gather(
                        disb, [jnp.broadcast_to(rl + 2 * p + half, (16,))]
                    )
                    sp = sp * scale
                    for f in range(FH // 16):
                        o = half * FH + f * 16
                        v = abuf[2 * p + half, pl.ds(f * 16, 16)] * sp
                        if recur:
                            v = v - tbuf[p, pl.ds(o, 16)]
                        obuf[p, pl.ds(o, 16)] = v
                return 0

            lax.fori_loop(0, RCH // 2, erow, 0)
            pltpu.sync_copy(obuf, out_hbm.at[c, pl.ds(p0, RCH // 2), :])

    return pl.kernel(
        body,
        out_type=jax.ShapeDtypeStruct((NC, NP // 2, F), jnp.float32),
        mesh=_mesh,
        scratch_types=[
            pltpu.VMEM_SHARED((NP, FH), jnp.float32),
            pltpu.VMEM_SHARED((NP, FH), jnp.float32),
            pltpu.VMEM((SUBG, B), jnp.int32),
            pltpu.VMEM((SUBG, B), jnp.int32),
            pltpu.VMEM((RCH // 2, F), jnp.float32),
            pltpu.VMEM((RCH, FH), jnp.float32),
            pltpu.VMEM((RCH, FH), jnp.float32),
            pltpu.VMEM((RCH // 2, F), jnp.float32),
            pltpu.VMEM((B, FH), jnp.float32),
            pltpu.VMEM((B, FH), jnp.float32),
            pltpu.VMEM((NPT,), jnp.float32),
            pltpu.SemaphoreType.DMA,
            pltpu.SemaphoreType.DMA,
            pltpu.SemaphoreType.DMA,
            pltpu.SemaphoreType.DMA,
        ],
        compiler_params=_sc_params,
    )


_prop_plain = _make_prop(False)
_prop_recur = _make_prop(True)


def _tc_layer(t0, t1, t2, t3, W, b, last):
    H = W.shape[2]
    BR = 512
    G = NP // BR

    def body(t0r, t1r, t2r, t3r, wr, br, outr):
        acc = br[...] * jnp.float32(1.0)
        for k, tr in enumerate((t0r, t1r, t2r, t3r)):
            acc = acc + jnp.dot(
                tr[...], wr[k], preferred_element_type=jnp.float32
            )
        if last:
            m = jnp.max(acc, axis=1, keepdims=True)
            z = acc - m
            lse = jnp.log(jnp.sum(jnp.exp(z), axis=1, keepdims=True))
            outr[...] = z - lse
        else:
            outr[...] = jnp.maximum(acc, 0.0)

    spec = pl.BlockSpec((BR, F), lambda i: (i, 0))
    return pl.pallas_call(
        body,
        out_shape=jax.ShapeDtypeStruct((NP, H), jnp.float32),
        grid=(G,),
        in_specs=[
            spec, spec, spec, spec,
            pl.BlockSpec((4, F, H), lambda i: (0, 0, 0)),
            pl.BlockSpec((1, H), lambda i: (0, 0)),
        ],
        out_specs=pl.BlockSpec((BR, H), lambda i: (i, 0)),
    )(_unsplit(t0), _unsplit(t1), _unsplit(t2), _unsplit(t3), W, b)


def _split(x2d):
    return jnp.stack(
        [x2d[:, :FH].reshape(NP // 2, F), x2d[:, FH:].reshape(NP // 2, F)]
    )


def _unsplit(slabs):
    return jnp.concatenate(
        [slabs[0].reshape(NP, FH), slabs[1].reshape(NP, FH)], axis=1
    )


def _pad_edges(idx):
    per = idx.reshape(NS, E // NS)
    fill = N + (jnp.arange(EPAD, dtype=jnp.int32) % NPAD)
    fill = jnp.broadcast_to(fill, (NS, EPAD))
    return jnp.concatenate([per, fill], axis=1).reshape(NS, EPT // B, B)


def kernel(x, edge_index, W1, b1, W2, b2, W3, b3):
    row2 = _pad_edges(edge_index[0])
    col2 = _pad_edges(edge_index[1])

    dis = _deg_dis(row2)
    xp = _split(jnp.pad(x, ((0, NP - N), (0, 0))))

    def cheb(t, W, b, last):
        t0 = t
        t1 = _prop_plain(t0, row2, col2, dis)
        t2 = _prop_recur(t1, t0, row2, col2, dis)
        t3 = _prop_recur(t2, t1, row2, col2, dis)
        return _tc_layer(t0, t1, t2, t3, W, b.reshape(1, -1), last)

    h = cheb(xp, W1, b1, False)
    h = cheb(_split(h), W2, b2, False)
    out = cheb(_split(h), W3, b3, True)
    return out[:N]
